# baseline (speedup 1.0000x reference)
import jax
import jax.numpy as jnp
from jax.experimental import pallas as pl
from jax.experimental.pallas import tpu as pltpu

_BLK = 16384


def _copy_body(in_hbm, out_ref, sem):
    i = pl.program_id(0)
    cp = pltpu.make_async_copy(
        in_hbm.at[pl.ds(i * _BLK, _BLK), :], out_ref, sem
    )
    cp.start()
    cp.wait()


def kernel(queue):
    rows, dim = queue.shape
    return pl.pallas_call(
        _copy_body,
        out_shape=jax.ShapeDtypeStruct(queue.shape, queue.dtype),
        grid=(rows // _BLK,),
        in_specs=[pl.BlockSpec(memory_space=pl.ANY)],
        out_specs=pl.BlockSpec((_BLK, dim), lambda i: (i, 0)),
        scratch_shapes=[pltpu.SemaphoreType.DMA],
        compiler_params=pltpu.CompilerParams(
            dimension_semantics=("arbitrary",),
        ),
    )(queue)


# speedup vs baseline: 1.2156x; 1.2156x over previous
import jax
import jax.numpy as jnp
from jax.experimental import pallas as pl
from jax.experimental.pallas import tpu as pltpu

_BLK = 16384


def _copy_body(in_ref, out_ref):
    out_ref[...] = in_ref[...]


def kernel(queue):
    rows, dim = queue.shape
    return pl.pallas_call(
        _copy_body,
        out_shape=jax.ShapeDtypeStruct(queue.shape, queue.dtype),
        grid=(rows // _BLK,),
        in_specs=[pl.BlockSpec((_BLK, dim), lambda i: (i, 0))],
        out_specs=pl.BlockSpec((_BLK, dim), lambda i: (i, 0)),
        compiler_params=pltpu.CompilerParams(
            dimension_semantics=("parallel",),
        ),
    )(queue)
